# speedup vs baseline: 4.2562x; 4.2562x over previous
import functools

import jax
import jax.numpy as jnp
from jax import lax
from jax.experimental import pallas as pl
from jax.experimental.pallas import tpu as pltpu
import jax.experimental.pallas.tpu_sc as plsc

P = 256
D = 128
N_SEG = 4096
E = 131072
BASE = 6.0

_NTILES = 16
_ROWS = E // 128
_ROWS_PER_TILE = _ROWS // _NTILES
_CHUNK_ROWS = 4
_NCHUNKS = _ROWS_PER_TILE // _CHUNK_ROWS
_SEG_PER_TILE = N_SEG // _NTILES


def _sc_msgs_body(drug_hbm, e0_hbm, e1_hbm, y_hbm, out_hbm,
                  acc, ei0_v, ei1_v, y_v, rows_v, sem):
    cid = lax.axis_index("c")
    sid = lax.axis_index("s")

    zeros16 = jnp.zeros((16,), jnp.float32)

    def zero_row(r, _):
        for q in range(8):
            rows_v[r, pl.ds(16 * q, 16)] = zeros16
        return 0

    lax.fori_loop(0, _SEG_PER_TILE, zero_row, 0)
    pltpu.sync_copy(rows_v.at[pl.ds(0, _SEG_PER_TILE)],
                    acc.at[pl.ds(sid * _SEG_PER_TILE, _SEG_PER_TILE)])
    plsc.subcore_barrier()

    def chunk_body(chunk, _):
        base = sid * _ROWS_PER_TILE + chunk * _CHUNK_ROWS
        ebase = base * 128
        pltpu.sync_copy(e1_hbm.at[cid, pl.ds(base, _CHUNK_ROWS)], ei1_v)
        pltpu.sync_copy(e0_hbm.at[cid, pl.ds(base, _CHUNK_ROWS)], ei0_v)
        pltpu.sync_copy(y_hbm.at[cid, pl.ds(ebase, _CHUNK_ROWS * 128)], y_v)

        copies = [
            pltpu.async_copy(drug_hbm.at[ei1_v.at[j]],
                             rows_v.at[pl.ds(j * 128, 128)], sem)
            for j in range(_CHUNK_ROWS)
        ]
        for c in copies:
            c.wait()

        def scale_group(g, _):
            y16 = y_v[pl.ds(g * 16, 16)] - BASE
            for l in range(16):
                c16 = jnp.full((16,), y16[l], jnp.float32)
                row = g * 16 + l
                for q in range(8):
                    sl = pl.ds(16 * q, 16)
                    rows_v[row, sl] = rows_v[row, sl] * c16
            return 0

        lax.fori_loop(0, _CHUNK_ROWS * 8, scale_group, 0)

        for j in range(_CHUNK_ROWS):
            pltpu.sync_copy(rows_v.at[pl.ds(j * 128, 128)],
                            acc.at[ei0_v.at[j]], add=True)
        return 0

    lax.fori_loop(0, _NCHUNKS, chunk_body, 0)
    plsc.subcore_barrier()

    pltpu.sync_copy(acc.at[pl.ds(sid * _SEG_PER_TILE, _SEG_PER_TILE)],
                    out_hbm.at[cid, pl.ds(sid * _SEG_PER_TILE, _SEG_PER_TILE)])


def _sc_msgs(drug_features, e0, e1, y):
    mesh = plsc.VectorSubcoreMesh(core_axis_name="c", subcore_axis_name="s")
    f = functools.partial(
        pl.kernel,
        mesh=mesh,
        out_type=jax.ShapeDtypeStruct((2, N_SEG, D), jnp.float32),
        scratch_types=[
            pltpu.VMEM_SHARED((N_SEG, D), jnp.float32),
            pltpu.VMEM((_CHUNK_ROWS, 128), jnp.int32),
            pltpu.VMEM((_CHUNK_ROWS, 128), jnp.int32),
            pltpu.VMEM((_CHUNK_ROWS * 128,), jnp.float32),
            pltpu.VMEM((_CHUNK_ROWS * 128, D), jnp.float32),
            pltpu.SemaphoreType.DMA,
        ],
    )(_sc_msgs_body)
    return f(drug_features, e0, e1, y)


def _tc_body(zt_ref, ff_ref, rf_ref, fmsg_ref, rmsg_ref, le_ref,
             w1a_ref, w1b_ref, w1c_ref, b1_ref, w2_ref, b2_ref,
             wm1_ref, bm1_ref, pw_ref, wm2_ref, bm2_ref, g_ref, b_ref,
             z_ref):
    zt = zt_ref[...]
    le = le_ref[...]
    w1a = w1a_ref[...]
    w1b = w1b_ref[...]
    w1c = w1c_ref[...]
    b1 = b1_ref[...]
    w2 = w2_ref[...]
    b2 = b2_ref[0, 0]

    c0 = jnp.dot(zt, w1a, preferred_element_type=jnp.float32) + b1
    c0f = c0 + jnp.dot(le[0:1], w1c, preferred_element_type=jnp.float32)
    c0r = c0 + jnp.dot(le[1:2], w1c, preferred_element_type=jnp.float32)

    def scores(feat, c0x):
        h = jnp.dot(feat, w1b, preferred_element_type=jnp.float32) + c0x
        h = jnp.where(h >= 0, h, 0.2 * h)
        return jnp.dot(h, w2, preferred_element_type=jnp.float32) + b2

    sf = scores(ff_ref[...], c0f)
    sr = scores(rf_ref[...], c0r)
    m = jnp.maximum(jnp.max(sf), jnp.max(sr))
    ef = jnp.exp(sf - m)
    er = jnp.exp(sr - m)
    z = jnp.sum(ef) + jnp.sum(er)

    vp = (jnp.dot(ef.T, fmsg_ref[...], preferred_element_type=jnp.float32)
          + jnp.dot(er.T, rmsg_ref[...], preferred_element_type=jnp.float32)) / z

    h1 = jnp.dot(vp, wm1_ref[...], preferred_element_type=jnp.float32) + bm1_ref[...]
    pw = pw_ref[0, 0]
    h1 = jnp.where(h1 >= 0, h1, pw * h1)
    out = jnp.dot(h1, wm2_ref[...], preferred_element_type=jnp.float32) + bm2_ref[...]

    x = zt + out
    mu = jnp.mean(x, axis=-1, keepdims=True)
    var = jnp.mean((x - mu) ** 2, axis=-1, keepdims=True)
    z_ref[...] = (x - mu) / jnp.sqrt(var + 1e-5) * g_ref[...] + b_ref[...]


def kernel(target_features, form_neighbors, form_binds_ei, form_binds_y,
           form_features, role_neighbors, role_binds_ei, role_binds_y,
           role_features, drug_features, layer_emb, W1, b1, W2, b2,
           Wm1, bm1, prelu_w, Wm2, bm2, ln_g, ln_b):
    e0 = jnp.stack([form_binds_ei[0].reshape(_ROWS, 128),
                    role_binds_ei[0].reshape(_ROWS, 128)])
    e1 = jnp.stack([form_binds_ei[1].reshape(_ROWS, 128),
                    role_binds_ei[1].reshape(_ROWS, 128)])
    y = jnp.stack([form_binds_y, role_binds_y])

    msgs = _sc_msgs(drug_features, e0, e1, y)
    form_msgs = msgs[0]
    role_msgs = msgs[1]

    z = pl.pallas_call(
        _tc_body,
        out_shape=jax.ShapeDtypeStruct((1, P), jnp.float32),
    )(target_features.reshape(1, P), form_features, role_features,
      form_msgs, role_msgs, layer_emb,
      W1[:P], W1[P:2 * P], W1[2 * P:], b1.reshape(1, 64),
      W2, b2.reshape(1, 1),
      Wm1, bm1.reshape(1, P), prelu_w.reshape(1, 1),
      Wm2, bm2.reshape(1, P), ln_g.reshape(1, P), ln_b.reshape(1, P))

    return (z.reshape(P), form_msgs, role_msgs)


# speedup vs baseline: 6.2646x; 1.4719x over previous
import functools

import jax
import jax.numpy as jnp
from jax import lax
from jax.experimental import pallas as pl
from jax.experimental.pallas import tpu as pltpu
import jax.experimental.pallas.tpu_sc as plsc

P = 256
D = 128
N_SEG = 4096
E = 131072
BASE = 6.0

_NTILES = 16
_ROWS = E // 128
_ROWS_PER_TILE = _ROWS // _NTILES
_CHUNK_ROWS = 2
_NCHUNKS = _ROWS_PER_TILE // _CHUNK_ROWS
_SEG_PER_TILE = N_SEG // _NTILES
_CE = _CHUNK_ROWS * 128


def _sc_msgs_body(drug_hbm, e0_hbm, e1_hbm, y_hbm, out_hbm,
                  acc, e0_v, e1_v, y_v, rows_a, rows_b, gsem, ssem):
    cid = lax.axis_index("c")
    sid = lax.axis_index("s")
    rbase = sid * _ROWS_PER_TILE

    pltpu.sync_copy(e1_hbm.at[cid, pl.ds(rbase, _ROWS_PER_TILE)], e1_v)
    pltpu.sync_copy(e0_hbm.at[cid, pl.ds(rbase, _ROWS_PER_TILE)], e0_v)
    pltpu.sync_copy(y_hbm.at[cid, pl.ds(rbase * 128, _ROWS_PER_TILE * 128)], y_v)

    zeros16 = jnp.zeros((16,), jnp.float32)

    def zero_row(r, _):
        for q in range(8):
            rows_a[r, pl.ds(16 * q, 16)] = zeros16
        return 0

    lax.fori_loop(0, _SEG_PER_TILE, zero_row, 0)
    pltpu.sync_copy(rows_a.at[pl.ds(0, _SEG_PER_TILE)],
                    acc.at[pl.ds(sid * _SEG_PER_TILE, _SEG_PER_TILE)])
    plsc.subcore_barrier()

    def fire_gather(buf, c):
        for j in range(_CHUNK_ROWS):
            pltpu.async_copy(drug_hbm.at[e1_v.at[c * _CHUNK_ROWS + j]],
                             buf.at[pl.ds(j * 128, 128)], gsem)

    def wait_gather(buf):
        for j in range(_CHUNK_ROWS):
            pltpu.make_async_copy(drug_hbm.at[e1_v.at[0]],
                                  buf.at[pl.ds(j * 128, 128)], gsem).wait()

    def fire_scatter(buf, c):
        for j in range(_CHUNK_ROWS):
            pltpu.async_copy(buf.at[pl.ds(j * 128, 128)],
                             acc.at[e0_v.at[c * _CHUNK_ROWS + j]], ssem,
                             add=True)

    def wait_scatter(buf):
        for j in range(_CHUNK_ROWS):
            pltpu.make_async_copy(buf.at[pl.ds(j * 128, 128)],
                                  acc.at[e0_v.at[0]], ssem).wait()

    def scale(buf, c):
        def scale_group(g, _):
            y16 = y_v[pl.ds(c * _CE + g * 16, 16)] - BASE
            for l in range(16):
                c16 = jnp.full((16,), y16[l], jnp.float32)
                row = g * 16 + l
                for q in range(8):
                    sl = pl.ds(16 * q, 16)
                    buf[row, sl] = buf[row, sl] * c16
            return 0

        lax.fori_loop(0, _CE // 16, scale_group, 0)

    fire_gather(rows_a, 0)

    def body(i, _):
        ca = 2 * i
        wait_gather(rows_a)

        @pl.when(i > 0)
        def _():
            wait_scatter(rows_b)

        fire_gather(rows_b, ca + 1)
        scale(rows_a, ca)
        fire_scatter(rows_a, ca)
        wait_gather(rows_b)
        wait_scatter(rows_a)

        @pl.when(i < _NCHUNKS // 2 - 1)
        def _():
            fire_gather(rows_a, ca + 2)

        scale(rows_b, ca + 1)
        fire_scatter(rows_b, ca + 1)
        return 0

    lax.fori_loop(0, _NCHUNKS // 2, body, 0)
    wait_scatter(rows_b)
    plsc.subcore_barrier()

    pltpu.sync_copy(acc.at[pl.ds(sid * _SEG_PER_TILE, _SEG_PER_TILE)],
                    out_hbm.at[cid, pl.ds(sid * _SEG_PER_TILE, _SEG_PER_TILE)])


def _sc_msgs(drug_features, e0, e1, y):
    mesh = plsc.VectorSubcoreMesh(core_axis_name="c", subcore_axis_name="s")
    f = functools.partial(
        pl.kernel,
        mesh=mesh,
        out_type=jax.ShapeDtypeStruct((2, N_SEG, D), jnp.float32),
        scratch_types=[
            pltpu.VMEM_SHARED((N_SEG, D), jnp.float32),
            pltpu.VMEM((_ROWS_PER_TILE, 128), jnp.int32),
            pltpu.VMEM((_ROWS_PER_TILE, 128), jnp.int32),
            pltpu.VMEM((_ROWS_PER_TILE * 128,), jnp.float32),
            pltpu.VMEM((_CE, D), jnp.float32),
            pltpu.VMEM((_CE, D), jnp.float32),
            pltpu.SemaphoreType.DMA,
            pltpu.SemaphoreType.DMA,
        ],
    )(_sc_msgs_body)
    return f(drug_features, e0, e1, y)


def _tc_body(zt_ref, ff_ref, rf_ref, fmsg_ref, rmsg_ref, le_ref,
             w1a_ref, w1b_ref, w1c_ref, b1_ref, w2_ref, b2_ref,
             wm1_ref, bm1_ref, pw_ref, wm2_ref, bm2_ref, g_ref, b_ref,
             z_ref):
    zt = zt_ref[...]
    le = le_ref[...]
    w1a = w1a_ref[...]
    w1b = w1b_ref[...]
    w1c = w1c_ref[...]
    b1 = b1_ref[...]
    w2 = w2_ref[...]
    b2 = b2_ref[0, 0]

    c0 = jnp.dot(zt, w1a, preferred_element_type=jnp.float32) + b1
    c0f = c0 + jnp.dot(le[0:1], w1c, preferred_element_type=jnp.float32)
    c0r = c0 + jnp.dot(le[1:2], w1c, preferred_element_type=jnp.float32)

    def scores(feat, c0x):
        h = jnp.dot(feat, w1b, preferred_element_type=jnp.float32) + c0x
        h = jnp.where(h >= 0, h, 0.2 * h)
        return jnp.dot(h, w2, preferred_element_type=jnp.float32) + b2

    sf = scores(ff_ref[...], c0f)
    sr = scores(rf_ref[...], c0r)
    m = jnp.maximum(jnp.max(sf), jnp.max(sr))
    ef = jnp.exp(sf - m)
    er = jnp.exp(sr - m)
    z = jnp.sum(ef) + jnp.sum(er)

    vp = (jnp.dot(ef.T, fmsg_ref[...], preferred_element_type=jnp.float32)
          + jnp.dot(er.T, rmsg_ref[...], preferred_element_type=jnp.float32)) / z

    h1 = jnp.dot(vp, wm1_ref[...], preferred_element_type=jnp.float32) + bm1_ref[...]
    pw = pw_ref[0, 0]
    h1 = jnp.where(h1 >= 0, h1, pw * h1)
    out = jnp.dot(h1, wm2_ref[...], preferred_element_type=jnp.float32) + bm2_ref[...]

    x = zt + out
    mu = jnp.mean(x, axis=-1, keepdims=True)
    var = jnp.mean((x - mu) ** 2, axis=-1, keepdims=True)
    z_ref[...] = (x - mu) / jnp.sqrt(var + 1e-5) * g_ref[...] + b_ref[...]


def kernel(target_features, form_neighbors, form_binds_ei, form_binds_y,
           form_features, role_neighbors, role_binds_ei, role_binds_y,
           role_features, drug_features, layer_emb, W1, b1, W2, b2,
           Wm1, bm1, prelu_w, Wm2, bm2, ln_g, ln_b):
    e0 = jnp.stack([form_binds_ei[0].reshape(_ROWS, 128),
                    role_binds_ei[0].reshape(_ROWS, 128)])
    e1 = jnp.stack([form_binds_ei[1].reshape(_ROWS, 128),
                    role_binds_ei[1].reshape(_ROWS, 128)])
    y = jnp.stack([form_binds_y, role_binds_y])

    msgs = _sc_msgs(drug_features, e0, e1, y)
    form_msgs = msgs[0]
    role_msgs = msgs[1]

    z = pl.pallas_call(
        _tc_body,
        out_shape=jax.ShapeDtypeStruct((1, P), jnp.float32),
    )(target_features.reshape(1, P), form_features, role_features,
      form_msgs, role_msgs, layer_emb,
      W1[:P], W1[P:2 * P], W1[2 * P:], b1.reshape(1, 64),
      W2, b2.reshape(1, 1),
      Wm1, bm1.reshape(1, P), prelu_w.reshape(1, 1),
      Wm2, bm2.reshape(1, P), ln_g.reshape(1, P), ln_b.reshape(1, P))

    return (z.reshape(P), form_msgs, role_msgs)
